# Initial kernel scaffold; baseline (speedup 1.0000x reference)
import jax
import jax.numpy as jnp
from jax.experimental import pallas as pl


def kernel(x, edge_index, edge_attr, batch, W_emb, We1, be1, W11, b11, g1, beta1, W12, b12, We2, be2, W21, b21, g2, beta2, W22, b22, Wf1, bf1, Wf2, bf2, Wb1, bb1, Wb2, bb2):
    raise NotImplementedError("write your pallas kernel here")


# speedup vs baseline: 3.5270x; 3.5270x over previous
import functools

import jax
import jax.numpy as jnp
from jax import lax
from jax.experimental import pallas as pl
from jax.experimental.pallas import tpu as pltpu
from jax.experimental.pallas import tpu_sc as plsc

NN = 10000
NE = 320000
NTOK = 5000
DD = 128
GG = 256

NCORE = 2
NSUB = 16
NWORK = NCORE * NSUB
LANES = 16

NPAD = 10240
RPW = NPAD // NWORK
GC = 80

EC = 80
EPW = NE // NWORK
NCHUNK = EPW // EC
RT = NPAD // NSUB
ZR = 128

RB = 1000
NB = NN // RB


def _emb_call(idx_pad, tbl):
    mesh = plsc.VectorSubcoreMesh(core_axis_name="c", subcore_axis_name="s")

    @functools.partial(
        pl.kernel,
        out_type=jax.ShapeDtypeStruct((NPAD, DD), jnp.float32),
        mesh=mesh,
        scratch_types=[
            pltpu.VMEM((1, GC), jnp.int32),
            pltpu.VMEM((GC, DD), jnp.float32),
        ],
    )
    def emb_k(idx_hbm, tbl_hbm, out_hbm, iref, rref):
        wid = lax.axis_index("s") * NCORE + lax.axis_index("c")

        @pl.loop(0, RPW // GC)
        def _(k):
            base = wid * RPW + k * GC
            pltpu.sync_copy(idx_hbm.at[pl.ds(base, GC)], iref.at[0])
            pltpu.sync_copy(tbl_hbm.at[iref.at[0]], rref)
            pltpu.sync_copy(rref, out_hbm.at[pl.ds(base, GC)])

    return emb_k(idx_pad, tbl)


def _edge_call(h, src, dst, ea, w, b):
    mesh = plsc.VectorSubcoreMesh(core_axis_name="c", subcore_axis_name="s")

    @functools.partial(
        pl.kernel,
        out_type=jax.ShapeDtypeStruct((NCORE, NPAD, DD), jnp.float32),
        mesh=mesh,
        scratch_types=[
            pltpu.VMEM((1, EC), jnp.int32),
            pltpu.VMEM((1, EC), jnp.int32),
            pltpu.VMEM((1, EC), jnp.float32),
            pltpu.VMEM((EC, DD), jnp.float32),
            pltpu.VMEM((ZR, DD), jnp.float32),
            pltpu.VMEM((1, DD), jnp.float32),
            pltpu.VMEM((1, DD), jnp.float32),
            pltpu.VMEM_SHARED((NPAD, DD), jnp.float32),
        ],
    )
    def edge_k(h_hbm, src_hbm, dst_hbm, ea_hbm, w_hbm, b_hbm, out_hbm,
               sref, dref, aref, rows, stg, wref, bref, acc):
        cid = lax.axis_index("c")
        sid = lax.axis_index("s")
        wid = sid * NCORE + cid

        pltpu.sync_copy(w_hbm, wref)
        pltpu.sync_copy(b_hbm, bref)
        wv = [wref[0, pl.ds(j * LANES, LANES)] for j in range(DD // LANES)]
        bv = [bref[0, pl.ds(j * LANES, LANES)] for j in range(DD // LANES)]

        @pl.loop(0, ZR)
        def _(i):
            for j in range(DD // LANES):
                stg[i, pl.ds(j * LANES, LANES)] = jnp.zeros((LANES,), jnp.float32)

        @pl.loop(0, RT // ZR)
        def _(i):
            pltpu.sync_copy(stg, acc.at[pl.ds(sid * RT + i * ZR, ZR)])

        plsc.subcore_barrier()

        @pl.loop(0, NCHUNK)
        def _(k):
            base = wid * EPW + k * EC
            pltpu.sync_copy(src_hbm.at[pl.ds(base, EC)], sref.at[0])
            pltpu.sync_copy(dst_hbm.at[pl.ds(base, EC)], dref.at[0])
            pltpu.sync_copy(ea_hbm.at[pl.ds(base, EC)], aref.at[0])
            pltpu.sync_copy(h_hbm.at[sref.at[0]], rows)

            @pl.loop(0, EC // LANES)
            def _(g):
                av = aref[0, pl.ds(g * LANES, LANES)]
                for l in range(LANES):
                    a = av[l]
                    e = g * LANES + l
                    for j in range(DD // LANES):
                        sl = pl.ds(j * LANES, LANES)
                        rows[e, sl] = jnp.maximum(
                            rows[e, sl] + a * wv[j] + bv[j], 0.0)

            pltpu.sync_copy(rows, acc.at[dref.at[0]], add=True)

        plsc.subcore_barrier()

        @pl.loop(0, RT // ZR)
        def _(i):
            r0 = sid * RT + i * ZR
            pltpu.sync_copy(acc.at[pl.ds(r0, ZR)], stg)
            pltpu.sync_copy(stg, out_hbm.at[cid].at[pl.ds(r0, ZR)])

    return edge_k(h, src, dst, ea, w, b)


def _mlp_stage1(h, parts, W1, b1):
    def body(h_ref, p_ref, W1_ref, b1_ref, y_ref, st_ref, accum):
        i = pl.program_id(0)

        @pl.when(i == 0)
        def _():
            accum[...] = jnp.zeros_like(accum)

        z = h_ref[...] + p_ref[0] + p_ref[1]
        y = jnp.dot(z, W1_ref[...], preferred_element_type=jnp.float32) + b1_ref[...]
        y_ref[...] = y
        s1 = jnp.sum(y, axis=0, keepdims=True)
        s2 = jnp.sum(y * y, axis=0, keepdims=True)
        accum[...] += jnp.concatenate([s1, s2], axis=0)
        st_ref[...] = accum[...]

    return pl.pallas_call(
        body,
        grid=(NB,),
        in_specs=[
            pl.BlockSpec((RB, DD), lambda i: (i, 0)),
            pl.BlockSpec((NCORE, RB, DD), lambda i: (0, i, 0)),
            pl.BlockSpec((DD, DD), lambda i: (0, 0)),
            pl.BlockSpec((1, DD), lambda i: (0, 0)),
        ],
        out_specs=[
            pl.BlockSpec((RB, DD), lambda i: (i, 0)),
            pl.BlockSpec((2, DD), lambda i: (0, 0)),
        ],
        out_shape=[
            jax.ShapeDtypeStruct((NN, DD), jnp.float32),
            jax.ShapeDtypeStruct((2, DD), jnp.float32),
        ],
        scratch_shapes=[pltpu.VMEM((2, DD), jnp.float32)],
    )(h, parts, W1, b1)


def _mlp_stage2(y, st, g, beta, W2, b2):
    def body(y_ref, st_ref, g_ref, be_ref, W2_ref, b2_ref, o_ref):
        mu = st_ref[0:1, :] * (1.0 / NN)
        var = st_ref[1:2, :] * (1.0 / NN) - mu * mu
        yn = (y_ref[...] - mu) * lax.rsqrt(var + 1e-5) * g_ref[...] + be_ref[...]
        z = jnp.maximum(yn, 0.0)
        o = jnp.dot(z, W2_ref[...], preferred_element_type=jnp.float32) + b2_ref[...]
        o_ref[...] = jnp.maximum(o, 0.0)

    return pl.pallas_call(
        body,
        grid=(NB,),
        in_specs=[
            pl.BlockSpec((RB, DD), lambda i: (i, 0)),
            pl.BlockSpec((2, DD), lambda i: (0, 0)),
            pl.BlockSpec((1, DD), lambda i: (0, 0)),
            pl.BlockSpec((1, DD), lambda i: (0, 0)),
            pl.BlockSpec((DD, DD), lambda i: (0, 0)),
            pl.BlockSpec((1, DD), lambda i: (0, 0)),
        ],
        out_specs=pl.BlockSpec((RB, DD), lambda i: (i, 0)),
        out_shape=jax.ShapeDtypeStruct((NN, DD), jnp.float32),
    )(y, st, g, beta, W2, b2)


def _pool_heads(batch2d, h1, h2, Wf1, bf1, Wf2, bf2, Wb1, bb1, Wb2, bb2):
    def body(b_ref, h1_ref, h2_ref, Wf1_ref, bf1_ref, Wf2_ref, bf2_ref,
             Wb1_ref, bb1_ref, Wb2_ref, bb2_ref, of_ref, ob_ref, p1a, p2a):
        i = pl.program_id(0)

        @pl.when(i == 0)
        def _():
            p1a[...] = jnp.zeros_like(p1a)
            p2a[...] = jnp.zeros_like(p2a)

        gids = lax.broadcasted_iota(jnp.int32, (GG, RB), 0)
        ohT = jnp.where(gids == b_ref[0], 1.0, 0.0)
        p1a[...] += jnp.dot(ohT, h1_ref[...], preferred_element_type=jnp.float32)
        p2a[...] += jnp.dot(ohT, h2_ref[...], preferred_element_type=jnp.float32)

        @pl.when(i == NB - 1)
        def _():
            hcat = jnp.concatenate([p1a[...], p2a[...]], axis=1)
            tf = jnp.maximum(
                jnp.dot(hcat, Wf1_ref[...], preferred_element_type=jnp.float32)
                + bf1_ref[...], 0.0)
            of_ref[...] = (
                jnp.dot(tf, Wf2_ref[...], preferred_element_type=jnp.float32)
                + bf2_ref[...])
            tb = jnp.maximum(
                jnp.dot(hcat, Wb1_ref[...], preferred_element_type=jnp.float32)
                + bb1_ref[...], 0.0)
            ob_ref[...] = (
                jnp.dot(tb, Wb2_ref[...], preferred_element_type=jnp.float32)
                + bb2_ref[...])

    return pl.pallas_call(
        body,
        grid=(NB,),
        in_specs=[
            pl.BlockSpec((1, 1, RB), lambda i: (i, 0, 0)),
            pl.BlockSpec((RB, DD), lambda i: (i, 0)),
            pl.BlockSpec((RB, DD), lambda i: (i, 0)),
            pl.BlockSpec((2 * DD, DD), lambda i: (0, 0)),
            pl.BlockSpec((1, DD), lambda i: (0, 0)),
            pl.BlockSpec((DD, NTOK), lambda i: (0, 0)),
            pl.BlockSpec((1, NTOK), lambda i: (0, 0)),
            pl.BlockSpec((2 * DD, DD), lambda i: (0, 0)),
            pl.BlockSpec((1, DD), lambda i: (0, 0)),
            pl.BlockSpec((DD, NTOK), lambda i: (0, 0)),
            pl.BlockSpec((1, NTOK), lambda i: (0, 0)),
        ],
        out_specs=[
            pl.BlockSpec((GG, NTOK), lambda i: (0, 0)),
            pl.BlockSpec((GG, NTOK), lambda i: (0, 0)),
        ],
        out_shape=[
            jax.ShapeDtypeStruct((GG, NTOK), jnp.float32),
            jax.ShapeDtypeStruct((GG, NTOK), jnp.float32),
        ],
        scratch_shapes=[
            pltpu.VMEM((GG, DD), jnp.float32),
            pltpu.VMEM((GG, DD), jnp.float32),
        ],
    )(batch2d, h1, h2, Wf1, bf1, Wf2, bf2, Wb1, bb1, Wb2, bb2)


def kernel(x, edge_index, edge_attr, batch, W_emb, We1, be1, W11, b11, g1,
           beta1, W12, b12, We2, be2, W21, b21, g2, beta2, W22, b22, Wf1,
           bf1, Wf2, bf2, Wb1, bb1, Wb2, bb2):
    xi = x[:, 0].astype(jnp.int32)
    idx_pad = jnp.concatenate([xi, jnp.zeros((NPAD - NN,), jnp.int32)])
    src = edge_index[0].astype(jnp.int32)
    dst = edge_index[1].astype(jnp.int32)
    ea = edge_attr[:, 0]

    h0 = _emb_call(idx_pad, W_emb)

    p1 = _edge_call(h0, src, dst, ea, We1, be1.reshape(1, DD))
    y1, st1 = _mlp_stage1(h0, p1, W11, b11.reshape(1, DD))
    h1 = _mlp_stage2(y1, st1, g1.reshape(1, DD), beta1.reshape(1, DD),
                     W12, b12.reshape(1, DD))

    p2 = _edge_call(h1, src, dst, ea, We2, be2.reshape(1, DD))
    y2, st2 = _mlp_stage1(h1, p2, W21, b21.reshape(1, DD))
    h2 = _mlp_stage2(y2, st2, g2.reshape(1, DD), beta2.reshape(1, DD),
                     W22, b22.reshape(1, DD))

    batch2d = batch.astype(jnp.int32).reshape(NB, 1, RB)
    lf, lb = _pool_heads(batch2d, h1, h2,
                         Wf1, bf1.reshape(1, DD), Wf2, bf2.reshape(1, NTOK),
                         Wb1, bb1.reshape(1, DD), Wb2, bb2.reshape(1, NTOK))
    return (lf, lb)
